# baseline (speedup 1.0000x reference)
import functools

import jax
import jax.numpy as jnp
from jax import lax
from jax.experimental import pallas as pl
from jax.experimental.pallas import tpu as pltpu
from jax.experimental.pallas import tpu_sc as plsc
from jax.experimental.layout import Format, Layout

def _sc_fmt():
    return Format(
        Layout(major_to_minor=(0, 1), tiling=((16,),)),
        jax.sharding.SingleDeviceSharding(jax.devices()[0]))

BATCH = 16384
HIDDEN = 64
NC = 2
NS = 16
NW = NC * NS
PER_W = BATCH // NW
CHUNK = 64
NCHUNK = PER_W // CHUNK
LANES = 16
GROUPS = CHUNK // LANES
MARGIN = 1.0


def _rsqrt16(x):
    x = jnp.maximum(x, jnp.float32(1e-12))
    i = plsc.bitcast(x, jnp.int32)
    y = plsc.bitcast(jnp.int32(0x5F3759DF) - (i >> 1), jnp.float32)
    for _ in range(3):
        y = y * (jnp.float32(1.5) - jnp.float32(0.5) * x * y * y)
    return y


def _body(ent_hbm, rel_hbm, nv_hbm,
          ph_hbm, pt_hbm, pr_hbm, nh_hbm, nt_hbm, nr_hbm,
          out_hbm,
          i_ph, i_pt, i_pr, i_nh, i_nt, i_nr,
          b_ph, b_pt, b_pr, b_pn, b_nh, b_nt, b_nr, b_nn,
          out_stage, sem_a, sem_b):
    wid = lax.axis_index("s") * NC + lax.axis_index("c")
    base = wid * PER_W

    for src, dst in ((ph_hbm, i_ph), (pt_hbm, i_pt), (pr_hbm, i_pr),
                     (nh_hbm, i_nh), (nt_hbm, i_nt), (nr_hbm, i_nr)):
        pltpu.sync_copy(src.at[pl.ds(base, PER_W)], dst)

    gathers = ((ent_hbm, i_ph, b_ph), (ent_hbm, i_pt, b_pt),
               (rel_hbm, i_pr, b_pr), (nv_hbm, i_pr, b_pn),
               (ent_hbm, i_nh, b_nh), (ent_hbm, i_nt, b_nt),
               (rel_hbm, i_nr, b_nr), (nv_hbm, i_nr, b_nn))
    sems = (sem_a, sem_b)

    def issue(g):
        b = g % 2
        descs = []
        for tbl, idx, buf in gathers:
            cp = pltpu.async_copy(
                tbl.at[idx.at[pl.ds(g * CHUNK, CHUNK)]], buf.at[b], sems[b])
            descs.append(cp)
        return descs

    pending = {0: issue(0)}
    loss_acc = jnp.zeros((LANES,), jnp.float32)

    for g in range(NCHUNK):
        if g + 1 < NCHUNK:
            pending[g + 1] = issue(g + 1)
        for cp in pending.pop(g):
            cp.wait()
        b = g % 2
        rp_h, rp_t, rp_r, rp_n = b_ph.at[b], b_pt.at[b], b_pr.at[b], b_pn.at[b]
        rn_h, rn_t, rn_r, rn_n = b_nh.at[b], b_nt.at[b], b_nr.at[b], b_nn.at[b]

        for g2 in range(GROUPS):
            row = lax.iota(jnp.int32, LANES) + jnp.int32(g2 * LANES)

            def pass1(j, acc):
                col = jnp.full((LANES,), j, jnp.int32)
                (phh, ptt, prr, pnn, phn, ptn,
                 qhh, qtt, qrr, qnn, qhn, qtn) = acc
                ph = plsc.load_gather(rp_h, [row, col])
                pt = plsc.load_gather(rp_t, [row, col])
                pr = plsc.load_gather(rp_r, [row, col])
                pn = plsc.load_gather(rp_n, [row, col])
                nh = plsc.load_gather(rn_h, [row, col])
                nt = plsc.load_gather(rn_t, [row, col])
                nr = plsc.load_gather(rn_r, [row, col])
                nn = plsc.load_gather(rn_n, [row, col])
                return (phh + ph * ph, ptt + pt * pt, prr + pr * pr,
                        pnn + pn * pn, phn + ph * pn, ptn + pt * pn,
                        qhh + nh * nh, qtt + nt * nt, qrr + nr * nr,
                        qnn + nn * nn, qhn + nh * nn, qtn + nt * nn)

            z = jnp.zeros((LANES,), jnp.float32)
            (phh, ptt, prr, pnn, phn, ptn,
             qhh, qtt, qrr, qnn, qhn, qtn) = lax.fori_loop(
                 0, HIDDEN, pass1, (z,) * 12, unroll=4)

            p_ih, p_it, p_ir = _rsqrt16(phh), _rsqrt16(ptt), _rsqrt16(prr)
            p_in = _rsqrt16(pnn)
            q_ih, q_it, q_ir = _rsqrt16(qhh), _rsqrt16(qtt), _rsqrt16(qrr)
            q_in = _rsqrt16(qnn)
            p_c = (phn * p_ih - ptn * p_it) * p_in * p_in
            q_c = (qhn * q_ih - qtn * q_it) * q_in * q_in

            def pass2(j, acc):
                col = jnp.full((LANES,), j, jnp.int32)
                accp, accn = acc
                ph = plsc.load_gather(rp_h, [row, col])
                pt = plsc.load_gather(rp_t, [row, col])
                pr = plsc.load_gather(rp_r, [row, col])
                pn = plsc.load_gather(rp_n, [row, col])
                nh = plsc.load_gather(rn_h, [row, col])
                nt = plsc.load_gather(rn_t, [row, col])
                nr = plsc.load_gather(rn_r, [row, col])
                nn = plsc.load_gather(rn_n, [row, col])
                vp = ph * p_ih + pr * p_ir - pt * p_it - p_c * pn
                vn = nh * q_ih + nr * q_ir - nt * q_it - q_c * nn
                return (accp + jnp.abs(vp), accn + jnp.abs(vn))

            accp, accn = lax.fori_loop(0, HIDDEN, pass2, (z, z), unroll=4)
            loss_acc = loss_acc + jnp.maximum(
                accp - accn + jnp.float32(MARGIN), jnp.float32(0.0))

    total = jnp.sum(loss_acc)
    out_stage[...] = jnp.full((LANES,), total, jnp.float32)
    pltpu.sync_copy(out_stage, out_hbm.at[wid])


@jax.jit
def _launch(ent, rel, nv, ph, pt, pr, nh, nt, nr):
    mesh = plsc.VectorSubcoreMesh(
        core_axis_name="c", subcore_axis_name="s",
        num_cores=NC, num_subcores=NS)
    fn = pl.kernel(
        _body,
        out_type=jax.ShapeDtypeStruct((NW, LANES), jnp.float32),
        mesh=mesh,
        compiler_params=pltpu.CompilerParams(needs_layout_passes=False,
                                             use_tc_tiling_on_sc=False),
        scratch_types=[pltpu.VMEM((PER_W,), jnp.int32)] * 6
        + [pltpu.VMEM((2, CHUNK, HIDDEN), jnp.float32)] * 8
        + [pltpu.VMEM((LANES,), jnp.float32),
           pltpu.SemaphoreType.DMA, pltpu.SemaphoreType.DMA],
    )
    return fn(ent, rel, nv, ph, pt, pr, nh, nt, nr)


def kernel(pos_h, pos_t, pos_r, neg_h, neg_t, neg_r,
           ent_embeddings, rel_embeddings, normal_vectors):
    i32 = jnp.int32
    fmt = _sc_fmt()
    ent_sc = jax.device_put(ent_embeddings, fmt)
    rel_sc = jax.device_put(rel_embeddings, fmt)
    nv_sc = jax.device_put(normal_vectors, fmt)
    partials = _launch(
        ent_sc, rel_sc, nv_sc,
        pos_h.astype(i32), pos_t.astype(i32), pos_r.astype(i32),
        neg_h.astype(i32), neg_t.astype(i32), neg_r.astype(i32))
    return jnp.sum(partials[:, 0])


# speedup vs baseline: 1.1356x; 1.1356x over previous
import functools

import jax
import jax.numpy as jnp
from jax import lax
from jax.experimental import pallas as pl
from jax.experimental.pallas import tpu as pltpu
from jax.experimental.pallas import tpu_sc as plsc

BATCH = 16384
HIDDEN = 64
PADW = 128
ENT_TOTAL = 1000000
REL_TOTAL = 1000
NC = 2
NS = 16
NW = NC * NS
PER_W = BATCH // NW
CHUNK = 16
NCHUNK = PER_W // CHUNK
LANES = 16
MARGIN = 1.0
F32 = jnp.float32
I32 = jnp.int32


def _rsqrt16(x):
    x = jnp.maximum(x, F32(1e-12))
    i = plsc.bitcast(x, I32)
    y = plsc.bitcast(I32(0x5F3759DF) - (i >> 1), F32)
    for _ in range(3):
        y = y * (F32(1.5) - F32(0.5) * x * y * y)
    return y


def _main_body(s_ent, s_rel, s_nv,
               ph_hbm, pt_hbm, pr_hbm, nh_hbm, nt_hbm, nr_hbm,
               out_hbm,
               i_ph, i_pt, i_pr, i_nh, i_nt, i_nr,
               k_ph, k_pt, k_pr, k_nh, k_nt, k_nr,
               b_ph, b_pt, b_nh, b_nt,
               b_pr, b_pn, b_nr, b_nn,
               t_ph, t_pt, t_pr, t_pn, t_nh, t_nt, t_nr, t_nn,
               out_stage, sem_a, sem_b):
    wid = lax.axis_index("s") * NC + lax.axis_index("c")
    base = wid * PER_W

    for src, dst in ((ph_hbm, i_ph), (pt_hbm, i_pt), (pr_hbm, i_pr),
                     (nh_hbm, i_nh), (nt_hbm, i_nt), (nr_hbm, i_nr)):
        pltpu.sync_copy(src.at[pl.ds(base, PER_W)], dst)

    def shift(k, _):
        sl = pl.ds(k * CHUNK, CHUNK)
        for i_r, k_r in ((i_ph, k_ph), (i_pt, k_pt), (i_pr, k_pr),
                         (i_nh, k_nh), (i_nt, k_nt), (i_nr, k_nr)):
            k_r[sl] = i_r[sl] >> 1
        return 0

    lax.fori_loop(0, NCHUNK, shift, 0, unroll=4)

    sems = (sem_a, sem_b)
    lanes = lax.iota(I32, LANES)
    lane_base = lanes * I32(PADW)
    bufs = (b_ph, b_pt, b_nh, b_nt, b_pr, b_pn, b_nr, b_nn)

    def copies(g, b):
        sem = sems[b]
        sl = pl.ds(g * CHUNK, CHUNK)
        srcs = (s_ent.at[k_ph.at[sl]], s_ent.at[k_pt.at[sl]],
                s_ent.at[k_nh.at[sl]], s_ent.at[k_nt.at[sl]],
                s_rel.at[k_pr.at[sl]], s_nv.at[k_pr.at[sl]],
                s_rel.at[k_nr.at[sl]], s_nv.at[k_nr.at[sl]])
        return [pltpu.make_async_copy(src, dst.at[b], sem)
                for src, dst in zip(srcs, bufs)]

    def issue(g, b):
        for cp in copies(g, b):
            cp.start()

    def compute(g, b):
        sl = pl.ds(g * CHUNK, CHUNK)
        a_ph = (i_ph[sl] & 1) << 6
        a_pt = (i_pt[sl] & 1) << 6
        a_nh = (i_nh[sl] & 1) << 6
        a_nt = (i_nt[sl] & 1) << 6
        a_pr = (i_pr[sl] & 1) << 6
        a_nr = (i_nr[sl] & 1) << 6
        rp_h, rp_t, rn_h, rn_t = b_ph.at[b], b_pt.at[b], b_nh.at[b], b_nt.at[b]
        rp_r, rp_n, rn_r, rn_n = b_pr.at[b], b_pn.at[b], b_nr.at[b], b_nn.at[b]

        def pass1(j, acc):
            (phh, ptt, prr, pnn, phn, ptn,
             qhh, qtt, qrr, qnn, qhn, qtn) = acc
            ph = plsc.load_gather(rp_h, [lanes, a_ph + j])
            pt = plsc.load_gather(rp_t, [lanes, a_pt + j])
            pr = plsc.load_gather(rp_r, [lanes, a_pr + j])
            pn = plsc.load_gather(rp_n, [lanes, a_pr + j])
            nh = plsc.load_gather(rn_h, [lanes, a_nh + j])
            nt = plsc.load_gather(rn_t, [lanes, a_nt + j])
            nr = plsc.load_gather(rn_r, [lanes, a_nr + j])
            nn = plsc.load_gather(rn_n, [lanes, a_nr + j])
            t_ph[j] = ph
            t_pt[j] = pt
            t_pr[j] = pr
            t_pn[j] = pn
            t_nh[j] = nh
            t_nt[j] = nt
            t_nr[j] = nr
            t_nn[j] = nn
            return (phh + ph * ph, ptt + pt * pt, prr + pr * pr,
                    pnn + pn * pn, phn + ph * pn, ptn + pt * pn,
                    qhh + nh * nh, qtt + nt * nt, qrr + nr * nr,
                    qnn + nn * nn, qhn + nh * nn, qtn + nt * nn)

        z = jnp.zeros((LANES,), F32)
        (phh, ptt, prr, pnn, phn, ptn,
         qhh, qtt, qrr, qnn, qhn, qtn) = lax.fori_loop(
             0, HIDDEN, pass1, (z,) * 12, unroll=8)

        p_ih, p_it, p_ir = _rsqrt16(phh), _rsqrt16(ptt), _rsqrt16(prr)
        p_in = _rsqrt16(pnn)
        q_ih, q_it, q_ir = _rsqrt16(qhh), _rsqrt16(qtt), _rsqrt16(qrr)
        q_in = _rsqrt16(qnn)
        p_c = (phn * p_ih - ptn * p_it) * p_in * p_in
        q_c = (qhn * q_ih - qtn * q_it) * q_in * q_in

        def pass2(j, acc):
            accp, accn = acc
            vp = (t_ph[j] * p_ih + t_pr[j] * p_ir
                  - t_pt[j] * p_it - p_c * t_pn[j])
            vn = (t_nh[j] * q_ih + t_nr[j] * q_ir
                  - t_nt[j] * q_it - q_c * t_nn[j])
            return (accp + jnp.abs(vp), accn + jnp.abs(vn))

        accp, accn = lax.fori_loop(0, HIDDEN, pass2, (z, z), unroll=8)
        return jnp.maximum(accp - accn + F32(MARGIN), F32(0.0))

    issue(0, 0)
    issue(1, 1)

    def pair(g2, loss):
        ga = g2 * 2
        for cp in copies(ga, 0):
            cp.wait()
        loss = loss + compute(ga, 0)

        @pl.when(ga + 2 < NCHUNK)
        def _():
            issue(ga + 2, 0)

        for cp in copies(ga + 1, 1):
            cp.wait()
        loss = loss + compute(ga + 1, 1)

        @pl.when(ga + 3 < NCHUNK)
        def _():
            issue(ga + 3, 1)
        return loss

    loss_acc = lax.fori_loop(0, NCHUNK // 2, pair, jnp.zeros((LANES,), F32))

    total = jnp.sum(loss_acc)
    out_stage[...] = jnp.where(lanes == 0, total, F32(0.0))
    pltpu.sync_copy(out_stage, out_hbm.at[pl.ds(wid * LANES, LANES)])


@jax.jit
def _launch(ent2, rel2, nv2, ph, pt, pr, nh, nt, nr):
    main = pl.kernel(
        _main_body,
        out_type=jax.ShapeDtypeStruct((NW * LANES,), F32),
        mesh=plsc.VectorSubcoreMesh(
            core_axis_name="c", subcore_axis_name="s",
            num_cores=NC, num_subcores=NS),
        compiler_params=pltpu.CompilerParams(needs_layout_passes=False,
                                             use_tc_tiling_on_sc=True),
        scratch_types=[pltpu.VMEM((PER_W,), I32)] * 12
        + [pltpu.VMEM((2, CHUNK, PADW), F32)] * 8
        + [pltpu.VMEM((HIDDEN, LANES), F32)] * 8
        + [pltpu.VMEM((LANES,), F32),
           pltpu.SemaphoreType.DMA, pltpu.SemaphoreType.DMA],
    )
    return main(ent2, rel2, nv2, ph, pt, pr, nh, nt, nr)


def kernel(pos_h, pos_t, pos_r, neg_h, neg_t, neg_r,
           ent_embeddings, rel_embeddings, normal_vectors):
    partials = _launch(
        ent_embeddings.reshape(ENT_TOTAL // 2, PADW),
        rel_embeddings.reshape(REL_TOTAL // 2, PADW),
        normal_vectors.reshape(REL_TOTAL // 2, PADW),
        pos_h.astype(I32), pos_t.astype(I32), pos_r.astype(I32),
        neg_h.astype(I32), neg_t.astype(I32), neg_r.astype(I32))
    return jnp.sum(partials)
